# baseline (speedup 1.0000x reference)
import jax
import jax.numpy as jnp
from jax import lax
from jax.experimental import pallas as pl
from jax.experimental.pallas import tpu as pltpu, tpu_sc as plsc

VOCAB = 1000000
DIM = 64
NREL = 64
B = 16384
NNEG = 1024

PDIM = 2 * DIM
NC = 2
NS = 16
NW = NC * NS
ROWS_W = B // NW
CHUNK = 128
NCHUNK = ROWS_W // CHUNK
HALF_W = ROWS_W // 2
NEG_W = NNEG // NW

BLK = 512
NOUT = 1 + 2 * NNEG


def _sc_gather_body(tbl2, rel2, lhs_idx, rhs_idx, rel_idx, neg_idx,
                    lhs_out, rhs_out, ops_out, neg_out,
                    lhs_iv, rhs_iv, rel_iv, neg_iv,
                    lhs_rv, rhs_rv, ops_rv, neg_rv, sem):
    wid = lax.axis_index("s") * NC + lax.axis_index("c")

    pltpu.sync_copy(lhs_idx.at[pl.ds(wid * NCHUNK, NCHUNK)], lhs_iv)
    pltpu.sync_copy(rhs_idx.at[pl.ds(wid * NCHUNK, NCHUNK)], rhs_iv)
    pltpu.sync_copy(rel_idx.at[pl.ds(wid * NCHUNK, NCHUNK)], rel_iv)
    pltpu.sync_copy(neg_idx.at[pl.ds(wid * NEG_W, NEG_W)], neg_iv)

    for r in range(2):
        copies = []
        for jj in range(NCHUNK // 2):
            j = r * (NCHUNK // 2) + jj
            dst = pl.ds(jj * CHUNK, CHUNK)
            copies.append(pltpu.async_copy(tbl2.at[lhs_iv.at[j]],
                                           lhs_rv.at[dst], sem))
            copies.append(pltpu.async_copy(tbl2.at[rhs_iv.at[j]],
                                           rhs_rv.at[dst], sem))
            copies.append(pltpu.async_copy(rel2.at[rel_iv.at[j]],
                                           ops_rv.at[dst], sem))
        if r == 0:
            copies.append(pltpu.async_copy(tbl2.at[neg_iv], neg_rv, sem))
        for c in copies:
            c.wait()
        half = pl.ds(wid * ROWS_W + r * HALF_W, HALF_W)
        pltpu.sync_copy(lhs_rv, lhs_out.at[half])
        pltpu.sync_copy(rhs_rv, rhs_out.at[half])
        pltpu.sync_copy(ops_rv, ops_out.at[half])
        if r == 0:
            pltpu.sync_copy(neg_rv, neg_out.at[pl.ds(wid * NEG_W, NEG_W)])


@jax.jit
def _sc_gather(tbl2, rel2, lhs_idx, rhs_idx, rel_idx, neg_idx):
    mesh = plsc.VectorSubcoreMesh(core_axis_name="c", subcore_axis_name="s",
                                  num_cores=NC, num_subcores=NS)
    f32 = jnp.float32
    return pl.kernel(
        _sc_gather_body,
        mesh=mesh,
        compiler_params=pltpu.CompilerParams(use_tc_tiling_on_sc=False),
        out_type=[
            jax.ShapeDtypeStruct((B, PDIM), f32),
            jax.ShapeDtypeStruct((B, PDIM), f32),
            jax.ShapeDtypeStruct((B, PDIM), f32),
            jax.ShapeDtypeStruct((NNEG, PDIM), f32),
        ],
        scratch_types=[
            pltpu.VMEM((NCHUNK, CHUNK), jnp.int32),
            pltpu.VMEM((NCHUNK, CHUNK), jnp.int32),
            pltpu.VMEM((NCHUNK, CHUNK), jnp.int32),
            pltpu.VMEM((NEG_W,), jnp.int32),
            pltpu.VMEM((HALF_W, PDIM), f32),
            pltpu.VMEM((HALF_W, PDIM), f32),
            pltpu.VMEM((HALF_W, PDIM), f32),
            pltpu.VMEM((NEG_W, PDIM), f32),
            pltpu.SemaphoreType.DMA,
        ],
    )(tbl2, rel2, lhs_idx, rhs_idx, rel_idx, neg_idx)


TC_C = 1024
NSUP = (VOCAB + 2 * TC_C - 1) // (2 * TC_C)
PROWS = NSUP * TC_C


def _tc_transpose_body(a_ref, b_ref, out_ref):
    eye = (lax.broadcasted_iota(jnp.int32, (DIM, DIM), 0) ==
           lax.broadcasted_iota(jnp.int32, (DIM, DIM), 1)
           ).astype(jnp.float32)
    dn = (((0,), (0,)), ((), ()))
    ta = lax.dot_general(a_ref[...], eye, dn,
                         preferred_element_type=jnp.float32)
    tb = lax.dot_general(b_ref[...], eye, dn,
                         preferred_element_type=jnp.float32)
    out_ref[...] = jnp.concatenate([ta, tb], axis=1)


@jax.jit
def _tc_transpose(tableT):
    grid = (NSUP,)
    return pl.pallas_call(
        _tc_transpose_body,
        grid=grid,
        in_specs=[
            pl.BlockSpec((DIM, TC_C), lambda i: (0, 2 * i)),
            pl.BlockSpec((DIM, TC_C),
                         lambda i: (0, jnp.minimum(2 * i + 1,
                                                   VOCAB // TC_C))),
        ],
        out_specs=pl.BlockSpec((TC_C, PDIM), lambda i: (i, 0)),
        out_shape=jax.ShapeDtypeStruct((PROWS, PDIM), jnp.float32),
    )(tableT, tableT)


def _sel_half(pair, mask):
    left = pair[:, :DIM]
    right = pair[:, DIM:]
    return left + mask * (right - left)


def _tc_score_body(lhs_ref, rhs_ref, ops_ref, ml_ref, mr_ref, mo_ref,
                   w2t_ref, out_ref):
    lhs = _sel_half(lhs_ref[...], ml_ref[...])
    ops = _sel_half(ops_ref[...], mo_ref[...])
    rhs_t = _sel_half(rhs_ref[...], mr_ref[...]) + ops
    lops = jnp.sum(lhs * ops, axis=1, keepdims=True)
    pdot = jnp.sum(lhs * (rhs_t - ops), axis=1, keepdims=True)
    x2 = jnp.concatenate([lhs, rhs_t, lops, pdot], axis=1)
    out_ref[...] = lax.dot_general(
        w2t_ref[...], x2, (((1,), (1,)), ((), ())),
        preferred_element_type=jnp.float32)


@jax.jit
def _tc_score(lhs2, rhs2, ops2, ml, mr, mo, w2t):
    grid = (B // BLK,)
    blk2 = pl.BlockSpec((BLK, PDIM), lambda i: (i, 0))
    blkm = pl.BlockSpec((BLK, DIM), lambda i: (i, 0))
    return pl.pallas_call(
        _tc_score_body,
        grid=grid,
        in_specs=[
            blk2, blk2, blk2, blkm, blkm, blkm,
            pl.BlockSpec((NOUT, 2 * DIM + 2), lambda i: (0, 0)),
        ],
        out_specs=pl.BlockSpec((NOUT, BLK), lambda i: (0, i)),
        out_shape=jax.ShapeDtypeStruct((NOUT, B), jnp.float32),
    )(lhs2, rhs2, ops2, ml, mr, mo, w2t)


def _mask(pred):
    return jnp.broadcast_to(
        pred.astype(jnp.float32)[:, None], (pred.shape[0], DIM))


def kernel(lhs_idx, rhs_idx, rel_idx, neg_idx, table, rel_ops):
    tbl2 = _tc_transpose(table.T)
    rel2 = rel_ops.reshape(NREL // 2, PDIM)
    li = lhs_idx.astype(jnp.int32)
    ri = rhs_idx.astype(jnp.int32)
    oi = rel_idx.astype(jnp.int32)
    ni = neg_idx.astype(jnp.int32)

    def _pack_row(idx):
        return (idx // (2 * TC_C)) * TC_C + (idx % (2 * TC_C)) % TC_C

    def _pack_hi(idx):
        return ((idx % (2 * TC_C)) >= TC_C)

    lhs_i2 = _pack_row(li).reshape(B // CHUNK, CHUNK)
    rhs_i2 = _pack_row(ri).reshape(B // CHUNK, CHUNK)
    rel_i2 = (oi // 2).reshape(B // CHUNK, CHUNK)
    neg_i = _pack_row(ni)

    lhs2, rhs2, ops2, neg2 = _sc_gather(tbl2, rel2, lhs_i2, rhs_i2,
                                        rel_i2, neg_i)

    ml = _mask(_pack_hi(li))
    mr = _mask(_pack_hi(ri))
    mo = _mask(oi % 2 == 1)
    neg = jnp.where(_pack_hi(ni)[:, None], neg2[:, DIM:], neg2[:, :DIM])

    zn64 = jnp.zeros((NNEG, DIM), jnp.float32)
    on = jnp.ones((NNEG, 1), jnp.float32)
    zn1 = jnp.zeros((NNEG, 1), jnp.float32)
    row0 = jnp.concatenate([jnp.zeros((1, 2 * DIM), jnp.float32),
                            jnp.ones((1, 2), jnp.float32)], axis=1)
    blk_rhsneg = jnp.concatenate([neg, zn64, on, zn1], axis=1)
    blk_lhsneg = jnp.concatenate([zn64, neg, zn1, zn1], axis=1)
    w2t = jnp.concatenate([row0, blk_rhsneg, blk_lhsneg], axis=0)

    return _tc_score(lhs2, rhs2, ops2, ml, mr, mo, w2t).T


# speedup vs baseline: 1.4738x; 1.4738x over previous
import jax
import jax.numpy as jnp
from jax import lax
from jax.experimental import pallas as pl
from jax.experimental.pallas import tpu as pltpu, tpu_sc as plsc

VOCAB = 1000000
DIM = 64
NREL = 64
B = 16384
NNEG = 1024

PDIM = 2 * DIM
NC = 2
NS = 16
NW = NC * NS
ROWS_W = B // NW
CHUNK = 128
NCHUNK = ROWS_W // CHUNK
HALF_W = ROWS_W // 2
NEG_W = NNEG // NW

BLK = 512
NOUT = 1 + 2 * NNEG


def _sc_gather_body(tbl2, rel2, lhs_idx, rhs_idx, rel_idx, neg_idx,
                    lhs_out, rhs_out, ops_out, neg_out,
                    lhs_iv, rhs_iv, rel_iv, neg_iv,
                    lhs_rv, rhs_rv, ops_rv, neg_rv, sem):
    wid = lax.axis_index("s") * NC + lax.axis_index("c")

    pltpu.sync_copy(lhs_idx.at[pl.ds(wid * NCHUNK, NCHUNK)], lhs_iv)
    pltpu.sync_copy(rhs_idx.at[pl.ds(wid * NCHUNK, NCHUNK)], rhs_iv)
    pltpu.sync_copy(rel_idx.at[pl.ds(wid * NCHUNK, NCHUNK)], rel_iv)
    pltpu.sync_copy(neg_idx.at[pl.ds(wid * NEG_W, NEG_W)], neg_iv)

    for r in range(2):
        copies = []
        for jj in range(NCHUNK // 2):
            j = r * (NCHUNK // 2) + jj
            dst = pl.ds(jj * CHUNK, CHUNK)
            copies.append(pltpu.async_copy(tbl2.at[lhs_iv.at[j]],
                                           lhs_rv.at[dst], sem))
            copies.append(pltpu.async_copy(tbl2.at[rhs_iv.at[j]],
                                           rhs_rv.at[dst], sem))
            copies.append(pltpu.async_copy(rel2.at[rel_iv.at[j]],
                                           ops_rv.at[dst], sem))
        if r == 0:
            copies.append(pltpu.async_copy(tbl2.at[neg_iv], neg_rv, sem))
        for c in copies:
            c.wait()
        half = pl.ds(wid * ROWS_W + r * HALF_W, HALF_W)
        pltpu.sync_copy(lhs_rv, lhs_out.at[half])
        pltpu.sync_copy(rhs_rv, rhs_out.at[half])
        pltpu.sync_copy(ops_rv, ops_out.at[half])
        if r == 0:
            pltpu.sync_copy(neg_rv, neg_out.at[pl.ds(wid * NEG_W, NEG_W)])


@jax.jit
def _sc_gather(tbl2, rel2, lhs_idx, rhs_idx, rel_idx, neg_idx):
    mesh = plsc.VectorSubcoreMesh(core_axis_name="c", subcore_axis_name="s",
                                  num_cores=NC, num_subcores=NS)
    f32 = jnp.float32
    return pl.kernel(
        _sc_gather_body,
        mesh=mesh,
        compiler_params=pltpu.CompilerParams(use_tc_tiling_on_sc=False),
        out_type=[
            jax.ShapeDtypeStruct((B, PDIM), f32),
            jax.ShapeDtypeStruct((B, PDIM), f32),
            jax.ShapeDtypeStruct((B, PDIM), f32),
            jax.ShapeDtypeStruct((NNEG, PDIM), f32),
        ],
        scratch_types=[
            pltpu.VMEM((NCHUNK, CHUNK), jnp.int32),
            pltpu.VMEM((NCHUNK, CHUNK), jnp.int32),
            pltpu.VMEM((NCHUNK, CHUNK), jnp.int32),
            pltpu.VMEM((NEG_W,), jnp.int32),
            pltpu.VMEM((HALF_W, PDIM), f32),
            pltpu.VMEM((HALF_W, PDIM), f32),
            pltpu.VMEM((HALF_W, PDIM), f32),
            pltpu.VMEM((NEG_W, PDIM), f32),
            pltpu.SemaphoreType.DMA,
        ],
    )(tbl2, rel2, lhs_idx, rhs_idx, rel_idx, neg_idx)


TC_C = 1024
NSUP = (VOCAB + 2 * TC_C - 1) // (2 * TC_C)
NSUP_PAD = ((NSUP + 3) // 4) * 4
PROWS = NSUP_PAD * TC_C


SBLK = 4


def _tc_transpose_body(x_ref, out_ref):
    eye = (lax.broadcasted_iota(jnp.int32, (DIM, DIM), 0) ==
           lax.broadcasted_iota(jnp.int32, (DIM, DIM), 1)
           ).astype(jnp.float32)
    dn = (((0,), (0,)), ((), ()))
    x = x_ref[...]
    parts = []
    for k in range(SBLK):
        a = x[:, 2 * k * TC_C:(2 * k + 1) * TC_C]
        b = x[:, (2 * k + 1) * TC_C:(2 * k + 2) * TC_C]
        ta = lax.dot_general(a, eye, dn, preferred_element_type=jnp.float32)
        tb = lax.dot_general(b, eye, dn, preferred_element_type=jnp.float32)
        parts.append(jnp.concatenate([ta, tb], axis=1))
    out_ref[...] = jnp.concatenate(parts, axis=0)


@jax.jit
def _tc_transpose(tableT):
    grid = (pl.cdiv(NSUP, SBLK),)
    return pl.pallas_call(
        _tc_transpose_body,
        grid=grid,
        in_specs=[pl.BlockSpec((DIM, SBLK * 2 * TC_C), lambda i: (0, i))],
        out_specs=pl.BlockSpec((SBLK * TC_C, PDIM), lambda i: (i, 0)),
        out_shape=jax.ShapeDtypeStruct((PROWS, PDIM), jnp.float32),
    )(tableT)


def _sel_half(pair, mask):
    return jnp.where(mask != 0.0, pair[:, DIM:], pair[:, :DIM])


def _tc_score_body(lhs_ref, rhs_ref, ops_ref, ml_ref, mr_ref, mo_ref,
                   w2t_ref, out_ref):
    lhs = _sel_half(lhs_ref[...], ml_ref[...])
    ops = _sel_half(ops_ref[...], mo_ref[...])
    rhs_t = _sel_half(rhs_ref[...], mr_ref[...]) + ops
    lops = jnp.sum(lhs * ops, axis=1, keepdims=True)
    pdot = jnp.sum(lhs * (rhs_t - ops), axis=1, keepdims=True)
    x2 = jnp.concatenate([lhs, rhs_t, lops, pdot], axis=1)
    out_ref[...] = lax.dot_general(
        w2t_ref[...], x2, (((1,), (1,)), ((), ())),
        preferred_element_type=jnp.float32)


@jax.jit
def _tc_score(lhs2, rhs2, ops2, ml, mr, mo, w2t):
    grid = (B // BLK,)
    blk2 = pl.BlockSpec((BLK, PDIM), lambda i: (i, 0))
    blkm = pl.BlockSpec((BLK, DIM), lambda i: (i, 0))
    return pl.pallas_call(
        _tc_score_body,
        grid=grid,
        in_specs=[
            blk2, blk2, blk2, blkm, blkm, blkm,
            pl.BlockSpec((NOUT, 2 * DIM + 2), lambda i: (0, 0)),
        ],
        out_specs=pl.BlockSpec((NOUT, BLK), lambda i: (0, i)),
        out_shape=jax.ShapeDtypeStruct((NOUT, B), jnp.float32),
    )(lhs2, rhs2, ops2, ml, mr, mo, w2t)


def _mask(pred):
    return jnp.broadcast_to(
        pred.astype(jnp.float32)[:, None], (pred.shape[0], DIM))


def kernel(lhs_idx, rhs_idx, rel_idx, neg_idx, table, rel_ops):
    tbl2 = _tc_transpose(table.T)
    rel2 = rel_ops.reshape(NREL // 2, PDIM)
    li = lhs_idx.astype(jnp.int32)
    ri = rhs_idx.astype(jnp.int32)
    oi = rel_idx.astype(jnp.int32)
    ni = neg_idx.astype(jnp.int32)

    def _pack_row(idx):
        return (idx // (2 * TC_C)) * TC_C + (idx % (2 * TC_C)) % TC_C

    def _pack_hi(idx):
        return ((idx % (2 * TC_C)) >= TC_C)

    lhs_i2 = _pack_row(li).reshape(B // CHUNK, CHUNK)
    rhs_i2 = _pack_row(ri).reshape(B // CHUNK, CHUNK)
    rel_i2 = (oi // 2).reshape(B // CHUNK, CHUNK)
    neg_i = _pack_row(ni)

    lhs2, rhs2, ops2, neg2 = _sc_gather(tbl2, rel2, lhs_i2, rhs_i2,
                                        rel_i2, neg_i)

    ml = _mask(_pack_hi(li))
    mr = _mask(_pack_hi(ri))
    mo = _mask(oi % 2 == 1)
    neg = jnp.where(_pack_hi(ni)[:, None], neg2[:, DIM:], neg2[:, :DIM])

    zn64 = jnp.zeros((NNEG, DIM), jnp.float32)
    on = jnp.ones((NNEG, 1), jnp.float32)
    zn1 = jnp.zeros((NNEG, 1), jnp.float32)
    row0 = jnp.concatenate([jnp.zeros((1, 2 * DIM), jnp.float32),
                            jnp.ones((1, 2), jnp.float32)], axis=1)
    blk_rhsneg = jnp.concatenate([neg, zn64, on, zn1], axis=1)
    blk_lhsneg = jnp.concatenate([zn64, neg, zn1, zn1], axis=1)
    w2t = jnp.concatenate([row0, blk_rhsneg, blk_lhsneg], axis=0)

    return _tc_score(lhs2, rhs2, ops2, ml, mr, mo, w2t).T


# speedup vs baseline: 1.7919x; 1.2159x over previous
import jax
import jax.numpy as jnp
from jax import lax
from jax.experimental import pallas as pl
from jax.experimental.pallas import tpu as pltpu, tpu_sc as plsc

VOCAB = 1000000
DIM = 64
NREL = 64
B = 16384
NNEG = 1024

PDIM = 2 * DIM
NC = 2
NS = 16
NW = NC * NS
ROWS_W = B // NW
CHUNK = 128
NCHUNK = ROWS_W // CHUNK
NEG_W = NNEG // NW

BLK = 512
NOUT = 1 + 2 * NNEG

TC_C = 1024
SBLK = 8
NSUP = (VOCAB + 2 * TC_C - 1) // (2 * TC_C)
NSUP_PAD = ((NSUP + SBLK - 1) // SBLK) * SBLK
PROWS = NSUP_PAD * TC_C


def _tc_pack_body(x_ref, out_ref):
    eye = (lax.broadcasted_iota(jnp.int32, (DIM, DIM), 0) ==
           lax.broadcasted_iota(jnp.int32, (DIM, DIM), 1)
           ).astype(jnp.float32)
    dn = (((0,), (0,)), ((), ()))
    x = x_ref[...]
    parts = []
    for k in range(SBLK):
        a = x[:, 2 * k * TC_C:(2 * k + 1) * TC_C]
        b = x[:, (2 * k + 1) * TC_C:(2 * k + 2) * TC_C]
        ta = lax.dot_general(a, eye, dn, preferred_element_type=jnp.float32)
        tb = lax.dot_general(b, eye, dn, preferred_element_type=jnp.float32)
        parts.append(jnp.concatenate([ta, tb], axis=1))
    out_ref[...] = jnp.concatenate(parts, axis=0)


@jax.jit
def _tc_pack(tableT):
    grid = (NSUP_PAD // SBLK,)
    return pl.pallas_call(
        _tc_pack_body,
        grid=grid,
        in_specs=[pl.BlockSpec((DIM, SBLK * 2 * TC_C), lambda i: (0, i))],
        out_specs=pl.BlockSpec((SBLK * TC_C, PDIM), lambda i: (i, 0)),
        out_shape=jax.ShapeDtypeStruct((PROWS, PDIM), jnp.float32),
    )(tableT)


def _sc_gather_body(tblv, relv, lhs_idx, rhs_idx, rel_idx, neg_idx,
                    lhs_out, rhs_out, ops_out, neg_out,
                    lhs_iv, rhs_iv, rel_iv, neg_iv,
                    lhs_rv, rhs_rv, ops_rv, neg_rv, sem):
    wid = lax.axis_index("s") * NC + lax.axis_index("c")
    base = wid * ROWS_W

    pltpu.sync_copy(lhs_idx.at[pl.ds(wid * NCHUNK, NCHUNK)], lhs_iv)
    pltpu.sync_copy(rhs_idx.at[pl.ds(wid * NCHUNK, NCHUNK)], rhs_iv)
    pltpu.sync_copy(rel_idx.at[pl.ds(wid * NCHUNK, NCHUNK)], rel_iv)
    pltpu.sync_copy(neg_idx.at[pl.ds(wid * NEG_W, NEG_W)], neg_iv)

    copies = []
    for j in range(NCHUNK):
        dst = pl.ds(j * CHUNK, CHUNK)
        copies.append(pltpu.async_copy(tblv.at[lhs_iv.at[j]],
                                       lhs_rv.at[dst], sem))
        copies.append(pltpu.async_copy(tblv.at[rhs_iv.at[j]],
                                       rhs_rv.at[dst], sem))
        copies.append(pltpu.async_copy(relv.at[rel_iv.at[j]],
                                       ops_rv.at[dst], sem))
    copies.append(pltpu.async_copy(tblv.at[neg_iv], neg_rv, sem))
    for c in copies:
        c.wait()

    pltpu.sync_copy(lhs_rv, lhs_out.at[pl.ds(base, ROWS_W)])
    pltpu.sync_copy(rhs_rv, rhs_out.at[pl.ds(base, ROWS_W)])
    pltpu.sync_copy(ops_rv, ops_out.at[pl.ds(base, ROWS_W)])
    pltpu.sync_copy(neg_rv, neg_out.at[pl.ds(wid * NEG_W, NEG_W)])


@jax.jit
def _sc_gather(tblv, relv, lhs_idx, rhs_idx, rel_idx, neg_idx):
    mesh = plsc.VectorSubcoreMesh(core_axis_name="c", subcore_axis_name="s",
                                  num_cores=NC, num_subcores=NS)
    f32 = jnp.float32
    return pl.kernel(
        _sc_gather_body,
        mesh=mesh,
        compiler_params=pltpu.CompilerParams(use_tc_tiling_on_sc=False),
        out_type=[
            jax.ShapeDtypeStruct((B, DIM), f32),
            jax.ShapeDtypeStruct((B, DIM), f32),
            jax.ShapeDtypeStruct((B, DIM), f32),
            jax.ShapeDtypeStruct((NNEG, DIM), f32),
        ],
        scratch_types=[
            pltpu.VMEM((NCHUNK, CHUNK), jnp.int32),
            pltpu.VMEM((NCHUNK, CHUNK), jnp.int32),
            pltpu.VMEM((NCHUNK, CHUNK), jnp.int32),
            pltpu.VMEM((NEG_W,), jnp.int32),
            pltpu.VMEM((ROWS_W, DIM), f32),
            pltpu.VMEM((ROWS_W, DIM), f32),
            pltpu.VMEM((ROWS_W, DIM), f32),
            pltpu.VMEM((NEG_W, DIM), f32),
            pltpu.SemaphoreType.DMA,
        ],
    )(tblv, relv, lhs_idx, rhs_idx, rel_idx, neg_idx)


def _tc_score_body(lhs_ref, rhs_ref, ops_ref, w2t_ref, out_ref):
    lhs = lhs_ref[...]
    ops = ops_ref[...]
    rhs_t = rhs_ref[...] + ops
    lops = jnp.sum(lhs * ops, axis=1, keepdims=True)
    pdot = jnp.sum(lhs * rhs_ref[...], axis=1, keepdims=True)
    x2 = jnp.concatenate([lhs, rhs_t, lops, pdot], axis=1)
    out_ref[...] = lax.dot_general(
        w2t_ref[...], x2, (((1,), (1,)), ((), ())),
        preferred_element_type=jnp.float32)


@jax.jit
def _tc_score(lhs, rhs, ops, w2t):
    grid = (B // BLK,)
    blk = pl.BlockSpec((BLK, DIM), lambda i: (i, 0))
    return pl.pallas_call(
        _tc_score_body,
        grid=grid,
        in_specs=[
            blk, blk, blk,
            pl.BlockSpec((NOUT, 2 * DIM + 2), lambda i: (0, 0)),
        ],
        out_specs=pl.BlockSpec((NOUT, BLK), lambda i: (0, i)),
        out_shape=jax.ShapeDtypeStruct((NOUT, B), jnp.float32),
    )(lhs, rhs, ops, w2t)


def kernel(lhs_idx, rhs_idx, rel_idx, neg_idx, table, rel_ops):
    tbl2 = _tc_pack(table.T)
    tblv = tbl2.reshape(2 * PROWS, DIM)

    li = lhs_idx.astype(jnp.int32)
    ri = rhs_idx.astype(jnp.int32)
    oi = rel_idx.astype(jnp.int32)
    ni = neg_idx.astype(jnp.int32)

    def _row(idx):
        sup, r = idx // (2 * TC_C), idx % (2 * TC_C)
        return 2 * (sup * TC_C + r % TC_C) + r // TC_C

    lhs_i2 = _row(li).reshape(B // CHUNK, CHUNK)
    rhs_i2 = _row(ri).reshape(B // CHUNK, CHUNK)
    rel_i2 = oi.reshape(B // CHUNK, CHUNK)
    neg_i = _row(ni)

    lhs, rhs, ops, neg = _sc_gather(tblv, rel_ops, lhs_i2, rhs_i2,
                                    rel_i2, neg_i)

    zn64 = jnp.zeros((NNEG, DIM), jnp.float32)
    on = jnp.ones((NNEG, 1), jnp.float32)
    zn1 = jnp.zeros((NNEG, 1), jnp.float32)
    row0 = jnp.concatenate([jnp.zeros((1, 2 * DIM), jnp.float32),
                            jnp.ones((1, 2), jnp.float32)], axis=1)
    blk_rhsneg = jnp.concatenate([neg, zn64, on, zn1], axis=1)
    blk_lhsneg = jnp.concatenate([zn64, neg, zn1, zn1], axis=1)
    w2t = jnp.concatenate([row0, blk_rhsneg, blk_lhsneg], axis=0)

    return _tc_score(lhs, rhs, ops, w2t).T


# speedup vs baseline: 2.0288x; 1.1322x over previous
import jax
import jax.numpy as jnp
from jax import lax
from jax.experimental import pallas as pl
from jax.experimental.pallas import tpu as pltpu, tpu_sc as plsc

VOCAB = 1000000
DIM = 64
NREL = 64
B = 16384
NNEG = 1024

PDIM = 2 * DIM
NC = 2
NS = 16
NW = NC * NS
ROWS_W = B // NW
CHUNK = 128
NCHUNK = ROWS_W // CHUNK
NEG_W = NNEG // NW

BLK = 1024
NOUT = 1 + 2 * NNEG

TC_C = 1024
SBLK = 8
NSUP = (VOCAB + 2 * TC_C - 1) // (2 * TC_C)
NSUP_PAD = ((NSUP + SBLK - 1) // SBLK) * SBLK
PROWS = NSUP_PAD * TC_C


def _tc_pack_body(x_ref, out_ref):
    r_i = lax.broadcasted_iota(jnp.int32, (DIM, PDIM), 0)
    c_i = lax.broadcasted_iota(jnp.int32, (DIM, PDIM), 1)
    eye_l = (c_i == r_i).astype(jnp.float32)
    eye_r = (c_i == r_i + DIM).astype(jnp.float32)
    dn = (((0,), (0,)), ((), ()))
    last = NSUP_PAD // SBLK - 1

    def compute(tail):
        x = x_ref[...]
        parts = []
        for k in range(SBLK):
            a = x[:, 2 * k * TC_C:(2 * k + 1) * TC_C]
            b = x[:, (2 * k + 1) * TC_C:(2 * k + 2) * TC_C]
            if tail:
                gbase = last * SBLK * 2 * TC_C
                a_valid = VOCAB - (gbase + 2 * k * TC_C)
                b_valid = VOCAB - (gbase + (2 * k + 1) * TC_C)
                if a_valid <= 0:
                    parts.append(jnp.zeros((TC_C, PDIM), jnp.float32))
                    continue
                if a_valid < TC_C:
                    lane = lax.broadcasted_iota(jnp.int32, (DIM, TC_C), 1)
                    a = jnp.where(lane < a_valid, a, 0.0)
                ta = lax.dot_general(a, eye_l, dn,
                                     preferred_element_type=jnp.float32)
                if b_valid <= 0:
                    parts.append(ta)
                    continue
                if b_valid < TC_C:
                    lane = lax.broadcasted_iota(jnp.int32, (DIM, TC_C), 1)
                    b = jnp.where(lane < b_valid, b, 0.0)
            else:
                ta = lax.dot_general(a, eye_l, dn,
                                     preferred_element_type=jnp.float32)
            tb = lax.dot_general(b, eye_r, dn,
                                 preferred_element_type=jnp.float32)
            parts.append(ta + tb)
        return jnp.concatenate(parts, axis=0)

    @pl.when(pl.program_id(0) != last)
    def _fast():
        out_ref[...] = compute(False)

    @pl.when(pl.program_id(0) == last)
    def _tail():
        out_ref[...] = compute(True)


@jax.jit
def _tc_pack(tableT):
    grid = (NSUP_PAD // SBLK,)
    return pl.pallas_call(
        _tc_pack_body,
        grid=grid,
        in_specs=[pl.BlockSpec((DIM, SBLK * 2 * TC_C), lambda i: (0, i))],
        out_specs=pl.BlockSpec((SBLK * TC_C, PDIM), lambda i: (i, 0)),
        out_shape=jax.ShapeDtypeStruct((PROWS, PDIM), jnp.float32),
    )(tableT)


def _sc_gather_body(tblv, relv, lhs_idx, rhs_idx, rel_idx, neg_idx,
                    lhs_out, rhs_out, ops_out, neg_out,
                    lhs_iv, rhs_iv, rel_iv, neg_iv,
                    lhs_rv, rhs_rv, ops_rv, neg_rv, sem):
    wid = lax.axis_index("s") * NC + lax.axis_index("c")
    base = wid * ROWS_W

    pltpu.sync_copy(lhs_idx.at[pl.ds(wid * NCHUNK, NCHUNK)], lhs_iv)
    pltpu.sync_copy(rhs_idx.at[pl.ds(wid * NCHUNK, NCHUNK)], rhs_iv)
    pltpu.sync_copy(rel_idx.at[pl.ds(wid * NCHUNK, NCHUNK)], rel_iv)
    pltpu.sync_copy(neg_idx.at[pl.ds(wid * NEG_W, NEG_W)], neg_iv)

    copies = []
    for j in range(NCHUNK):
        dst = pl.ds(j * CHUNK, CHUNK)
        copies.append(pltpu.async_copy(tblv.at[lhs_iv.at[j]],
                                       lhs_rv.at[dst], sem))
        copies.append(pltpu.async_copy(tblv.at[rhs_iv.at[j]],
                                       rhs_rv.at[dst], sem))
        copies.append(pltpu.async_copy(relv.at[rel_iv.at[j]],
                                       ops_rv.at[dst], sem))
    copies.append(pltpu.async_copy(tblv.at[neg_iv], neg_rv, sem))
    for c in copies:
        c.wait()

    pltpu.sync_copy(lhs_rv, lhs_out.at[pl.ds(base, ROWS_W)])
    pltpu.sync_copy(rhs_rv, rhs_out.at[pl.ds(base, ROWS_W)])
    pltpu.sync_copy(ops_rv, ops_out.at[pl.ds(base, ROWS_W)])
    pltpu.sync_copy(neg_rv, neg_out.at[pl.ds(wid * NEG_W, NEG_W)])


@jax.jit
def _sc_gather(tblv, relv, lhs_idx, rhs_idx, rel_idx, neg_idx):
    mesh = plsc.VectorSubcoreMesh(core_axis_name="c", subcore_axis_name="s",
                                  num_cores=NC, num_subcores=NS)
    f32 = jnp.float32
    return pl.kernel(
        _sc_gather_body,
        mesh=mesh,
        compiler_params=pltpu.CompilerParams(use_tc_tiling_on_sc=False),
        out_type=[
            jax.ShapeDtypeStruct((B, DIM), f32),
            jax.ShapeDtypeStruct((B, DIM), f32),
            jax.ShapeDtypeStruct((B, DIM), f32),
            jax.ShapeDtypeStruct((NNEG, DIM), f32),
        ],
        scratch_types=[
            pltpu.VMEM((NCHUNK, CHUNK), jnp.int32),
            pltpu.VMEM((NCHUNK, CHUNK), jnp.int32),
            pltpu.VMEM((NCHUNK, CHUNK), jnp.int32),
            pltpu.VMEM((NEG_W,), jnp.int32),
            pltpu.VMEM((ROWS_W, DIM), f32),
            pltpu.VMEM((ROWS_W, DIM), f32),
            pltpu.VMEM((ROWS_W, DIM), f32),
            pltpu.VMEM((NEG_W, DIM), f32),
            pltpu.SemaphoreType.DMA,
        ],
    )(tblv, relv, lhs_idx, rhs_idx, rel_idx, neg_idx)


def _tc_score_body(lhs_ref, rhs_ref, ops_ref, w2t_ref, out_ref):
    lhs = lhs_ref[...]
    ops = ops_ref[...]
    rhs_t = rhs_ref[...] + ops
    lops = jnp.sum(lhs * ops, axis=1, keepdims=True)
    pdot = jnp.sum(lhs * rhs_ref[...], axis=1, keepdims=True)
    x2 = jnp.concatenate([lhs, rhs_t, lops, pdot], axis=1)
    out_ref[...] = lax.dot_general(
        w2t_ref[...], x2, (((1,), (1,)), ((), ())),
        preferred_element_type=jnp.float32)


@jax.jit
def _tc_score(lhs, rhs, ops, w2t):
    grid = (B // BLK,)
    blk = pl.BlockSpec((BLK, DIM), lambda i: (i, 0))
    return pl.pallas_call(
        _tc_score_body,
        grid=grid,
        in_specs=[
            blk, blk, blk,
            pl.BlockSpec((NOUT, 2 * DIM + 2), lambda i: (0, 0)),
        ],
        out_specs=pl.BlockSpec((NOUT, BLK), lambda i: (0, i)),
        out_shape=jax.ShapeDtypeStruct((NOUT, B), jnp.float32),
    )(lhs, rhs, ops, w2t)


def kernel(lhs_idx, rhs_idx, rel_idx, neg_idx, table, rel_ops):
    tbl2 = _tc_pack(table.T)
    tblv = tbl2.reshape(2 * PROWS, DIM)

    li = lhs_idx.astype(jnp.int32)
    ri = rhs_idx.astype(jnp.int32)
    oi = rel_idx.astype(jnp.int32)
    ni = neg_idx.astype(jnp.int32)

    def _row(idx):
        sup, r = idx // (2 * TC_C), idx % (2 * TC_C)
        return 2 * (sup * TC_C + r % TC_C) + r // TC_C

    lhs_i2 = _row(li).reshape(B // CHUNK, CHUNK)
    rhs_i2 = _row(ri).reshape(B // CHUNK, CHUNK)
    rel_i2 = oi.reshape(B // CHUNK, CHUNK)
    neg_i = _row(ni)

    lhs, rhs, ops, neg = _sc_gather(tblv, rel_ops, lhs_i2, rhs_i2,
                                    rel_i2, neg_i)

    zn64 = jnp.zeros((NNEG, DIM), jnp.float32)
    on = jnp.ones((NNEG, 1), jnp.float32)
    zn1 = jnp.zeros((NNEG, 1), jnp.float32)
    row0 = jnp.concatenate([jnp.zeros((1, 2 * DIM), jnp.float32),
                            jnp.ones((1, 2), jnp.float32)], axis=1)
    blk_rhsneg = jnp.concatenate([neg, zn64, on, zn1], axis=1)
    blk_lhsneg = jnp.concatenate([zn64, neg, zn1, zn1], axis=1)
    w2t = jnp.concatenate([row0, blk_rhsneg, blk_lhsneg], axis=0)

    return _tc_score(lhs, rhs, ops, w2t).T


# speedup vs baseline: 2.1734x; 1.0713x over previous
import jax
import jax.numpy as jnp
from jax import lax
from jax.experimental import pallas as pl
from jax.experimental.pallas import tpu as pltpu, tpu_sc as plsc

VOCAB = 1000000
DIM = 64
NREL = 64
B = 16384
NNEG = 1024

PDIM = 2 * DIM
NC = 2
NS = 16
NW = NC * NS
ROWS_W = B // NW
CHUNK = 128
NCHUNK = ROWS_W // CHUNK
NEG_W = NNEG // NW

BLK = 1024
NOUT = 1 + 2 * NNEG

TC_C = 1024
SBLK = 16
NSUP = (VOCAB + 2 * TC_C - 1) // (2 * TC_C)
NSUP_PAD = ((NSUP + SBLK - 1) // SBLK) * SBLK
PROWS = NSUP_PAD * TC_C


def _tc_pack_body(x_ref, out_ref):
    r_i = lax.broadcasted_iota(jnp.int32, (DIM, PDIM), 0)
    c_i = lax.broadcasted_iota(jnp.int32, (DIM, PDIM), 1)
    eye_l = (c_i == r_i).astype(jnp.float32)
    eye_r = (c_i == r_i + DIM).astype(jnp.float32)
    dn = (((0,), (0,)), ((), ()))
    last = NSUP_PAD // SBLK - 1

    def compute(tail):
        x = x_ref[...]
        parts = []
        for k in range(SBLK):
            a = x[:, 2 * k * TC_C:(2 * k + 1) * TC_C]
            b = x[:, (2 * k + 1) * TC_C:(2 * k + 2) * TC_C]
            if tail:
                gbase = last * SBLK * 2 * TC_C
                a_valid = VOCAB - (gbase + 2 * k * TC_C)
                b_valid = VOCAB - (gbase + (2 * k + 1) * TC_C)
                if a_valid <= 0:
                    parts.append(jnp.zeros((TC_C, PDIM), jnp.float32))
                    continue
                if a_valid < TC_C:
                    lane = lax.broadcasted_iota(jnp.int32, (DIM, TC_C), 1)
                    a = jnp.where(lane < a_valid, a, 0.0)
                ta = lax.dot_general(a, eye_l, dn,
                                     preferred_element_type=jnp.float32)
                if b_valid <= 0:
                    parts.append(ta)
                    continue
                if b_valid < TC_C:
                    lane = lax.broadcasted_iota(jnp.int32, (DIM, TC_C), 1)
                    b = jnp.where(lane < b_valid, b, 0.0)
            else:
                ta = lax.dot_general(a, eye_l, dn,
                                     preferred_element_type=jnp.float32)
            tb = lax.dot_general(b, eye_r, dn,
                                 preferred_element_type=jnp.float32)
            parts.append(ta + tb)
        return jnp.concatenate(parts, axis=0)

    @pl.when(pl.program_id(0) != last)
    def _fast():
        out_ref[...] = compute(False)

    @pl.when(pl.program_id(0) == last)
    def _tail():
        out_ref[...] = compute(True)


@jax.jit
def _tc_pack(tableT):
    grid = (NSUP_PAD // SBLK,)
    return pl.pallas_call(
        _tc_pack_body,
        grid=grid,
        in_specs=[pl.BlockSpec((DIM, SBLK * 2 * TC_C), lambda i: (0, i))],
        out_specs=pl.BlockSpec((SBLK * TC_C, PDIM), lambda i: (i, 0)),
        out_shape=jax.ShapeDtypeStruct((PROWS, PDIM), jnp.float32),
    )(tableT)


def _sc_gather_body(tblv, relv, lhs_idx, rhs_idx, rel_idx, neg_idx,
                    lhs_out, rhs_out, ops_out, neg_out,
                    lhs_iv, rhs_iv, rel_iv, neg_iv,
                    lhs_rv, rhs_rv, ops_rv, neg_rv, sem):
    wid = lax.axis_index("s") * NC + lax.axis_index("c")
    base = wid * ROWS_W

    pltpu.sync_copy(lhs_idx.at[pl.ds(wid * NCHUNK, NCHUNK)], lhs_iv)
    pltpu.sync_copy(rhs_idx.at[pl.ds(wid * NCHUNK, NCHUNK)], rhs_iv)
    pltpu.sync_copy(rel_idx.at[pl.ds(wid * NCHUNK, NCHUNK)], rel_iv)
    pltpu.sync_copy(neg_idx.at[pl.ds(wid * NEG_W, NEG_W)], neg_iv)

    copies = []
    for j in range(NCHUNK):
        dst = pl.ds(j * CHUNK, CHUNK)
        copies.append(pltpu.async_copy(tblv.at[lhs_iv.at[j]],
                                       lhs_rv.at[dst], sem))
        copies.append(pltpu.async_copy(tblv.at[rhs_iv.at[j]],
                                       rhs_rv.at[dst], sem))
        copies.append(pltpu.async_copy(relv.at[rel_iv.at[j]],
                                       ops_rv.at[dst], sem))
    copies.append(pltpu.async_copy(tblv.at[neg_iv], neg_rv, sem))
    for c in copies:
        c.wait()

    pltpu.sync_copy(lhs_rv, lhs_out.at[pl.ds(base, ROWS_W)])
    pltpu.sync_copy(rhs_rv, rhs_out.at[pl.ds(base, ROWS_W)])
    pltpu.sync_copy(ops_rv, ops_out.at[pl.ds(base, ROWS_W)])
    pltpu.sync_copy(neg_rv, neg_out.at[pl.ds(wid * NEG_W, NEG_W)])


@jax.jit
def _sc_gather(tblv, relv, lhs_idx, rhs_idx, rel_idx, neg_idx):
    mesh = plsc.VectorSubcoreMesh(core_axis_name="c", subcore_axis_name="s",
                                  num_cores=NC, num_subcores=NS)
    f32 = jnp.float32
    return pl.kernel(
        _sc_gather_body,
        mesh=mesh,
        compiler_params=pltpu.CompilerParams(use_tc_tiling_on_sc=False),
        out_type=[
            jax.ShapeDtypeStruct((B, DIM), f32),
            jax.ShapeDtypeStruct((B, DIM), f32),
            jax.ShapeDtypeStruct((B, DIM), f32),
            jax.ShapeDtypeStruct((NNEG, DIM), f32),
        ],
        scratch_types=[
            pltpu.VMEM((NCHUNK, CHUNK), jnp.int32),
            pltpu.VMEM((NCHUNK, CHUNK), jnp.int32),
            pltpu.VMEM((NCHUNK, CHUNK), jnp.int32),
            pltpu.VMEM((NEG_W,), jnp.int32),
            pltpu.VMEM((ROWS_W, DIM), f32),
            pltpu.VMEM((ROWS_W, DIM), f32),
            pltpu.VMEM((ROWS_W, DIM), f32),
            pltpu.VMEM((NEG_W, DIM), f32),
            pltpu.SemaphoreType.DMA,
        ],
    )(tblv, relv, lhs_idx, rhs_idx, rel_idx, neg_idx)


def _tc_score_body(lhs_ref, rhs_ref, ops_ref, w2t_ref, out_ref):
    lhs = lhs_ref[...]
    ops = ops_ref[...]
    rhs_t = rhs_ref[...] + ops
    lops = jnp.sum(lhs * ops, axis=1, keepdims=True)
    pdot = jnp.sum(lhs * rhs_ref[...], axis=1, keepdims=True)
    x2 = jnp.concatenate([lhs, rhs_t, lops, pdot], axis=1)
    out_ref[...] = lax.dot_general(
        w2t_ref[...], x2, (((1,), (1,)), ((), ())),
        preferred_element_type=jnp.float32)


@jax.jit
def _tc_score(lhs, rhs, ops, w2t):
    grid = (B // BLK,)
    blk = pl.BlockSpec((BLK, DIM), lambda i: (i, 0))
    return pl.pallas_call(
        _tc_score_body,
        grid=grid,
        in_specs=[
            blk, blk, blk,
            pl.BlockSpec((NOUT, 2 * DIM + 2), lambda i: (0, 0)),
        ],
        out_specs=pl.BlockSpec((NOUT, BLK), lambda i: (0, i)),
        out_shape=jax.ShapeDtypeStruct((NOUT, B), jnp.float32),
    )(lhs, rhs, ops, w2t)


def kernel(lhs_idx, rhs_idx, rel_idx, neg_idx, table, rel_ops):
    tbl2 = _tc_pack(table.T)
    tblv = tbl2.reshape(2 * PROWS, DIM)

    li = lhs_idx.astype(jnp.int32)
    ri = rhs_idx.astype(jnp.int32)
    oi = rel_idx.astype(jnp.int32)
    ni = neg_idx.astype(jnp.int32)

    def _row(idx):
        sup, r = idx // (2 * TC_C), idx % (2 * TC_C)
        return 2 * (sup * TC_C + r % TC_C) + r // TC_C

    lhs_i2 = _row(li).reshape(B // CHUNK, CHUNK)
    rhs_i2 = _row(ri).reshape(B // CHUNK, CHUNK)
    rel_i2 = oi.reshape(B // CHUNK, CHUNK)
    neg_i = _row(ni)

    lhs, rhs, ops, neg = _sc_gather(tblv, rel_ops, lhs_i2, rhs_i2,
                                    rel_i2, neg_i)

    zn64 = jnp.zeros((NNEG, DIM), jnp.float32)
    on = jnp.ones((NNEG, 1), jnp.float32)
    zn1 = jnp.zeros((NNEG, 1), jnp.float32)
    row0 = jnp.concatenate([jnp.zeros((1, 2 * DIM), jnp.float32),
                            jnp.ones((1, 2), jnp.float32)], axis=1)
    blk_rhsneg = jnp.concatenate([neg, zn64, on, zn1], axis=1)
    blk_lhsneg = jnp.concatenate([zn64, neg, zn1, zn1], axis=1)
    w2t = jnp.concatenate([row0, blk_rhsneg, blk_lhsneg], axis=0)

    return _tc_score(lhs, rhs, ops, w2t).T


# speedup vs baseline: 2.1861x; 1.0058x over previous
import jax
import jax.numpy as jnp
from jax import lax
from jax.experimental import pallas as pl
from jax.experimental.pallas import tpu as pltpu, tpu_sc as plsc

VOCAB = 1000000
DIM = 64
NREL = 64
B = 16384
NNEG = 1024

PDIM = 2 * DIM
NC = 2
NS = 16
NW = NC * NS
ROWS_W = B // NW
CHUNK = 128
NCHUNK = ROWS_W // CHUNK
NEG_W = NNEG // NW

BLK = 1024
NOUT = 1 + 2 * NNEG

TC_C = 1024
SBLK = 16
NSUP = (VOCAB + 2 * TC_C - 1) // (2 * TC_C)
NSUP_PAD = ((NSUP + SBLK - 1) // SBLK) * SBLK
PROWS = NSUP_PAD * TC_C


def _tc_pack_body(x_ref, out_ref):
    r_i = lax.broadcasted_iota(jnp.int32, (DIM, PDIM), 0)
    c_i = lax.broadcasted_iota(jnp.int32, (DIM, PDIM), 1)
    eye_l = (c_i == r_i).astype(jnp.float32)
    eye_r = (c_i == r_i + DIM).astype(jnp.float32)
    dn = (((0,), (0,)), ((), ()))
    last = NSUP_PAD // SBLK - 1

    def compute(tail):
        x = x_ref[...]
        parts = []
        for k in range(SBLK):
            a = x[:, 2 * k * TC_C:(2 * k + 1) * TC_C]
            b = x[:, (2 * k + 1) * TC_C:(2 * k + 2) * TC_C]
            if tail:
                gbase = last * SBLK * 2 * TC_C
                a_valid = VOCAB - (gbase + 2 * k * TC_C)
                b_valid = VOCAB - (gbase + (2 * k + 1) * TC_C)
                if a_valid <= 0:
                    parts.append(jnp.zeros((TC_C, PDIM), jnp.float32))
                    continue
                if a_valid < TC_C:
                    lane = lax.broadcasted_iota(jnp.int32, (DIM, TC_C), 1)
                    a = jnp.where(lane < a_valid, a, 0.0)
                ta = lax.dot_general(a, eye_l, dn,
                                     preferred_element_type=jnp.float32)
                if b_valid <= 0:
                    parts.append(ta)
                    continue
                if b_valid < TC_C:
                    lane = lax.broadcasted_iota(jnp.int32, (DIM, TC_C), 1)
                    b = jnp.where(lane < b_valid, b, 0.0)
            else:
                ta = lax.dot_general(a, eye_l, dn,
                                     preferred_element_type=jnp.float32)
            tb = lax.dot_general(b, eye_r, dn,
                                 preferred_element_type=jnp.float32)
            parts.append(ta + tb)
        return jnp.concatenate(parts, axis=0)

    @pl.when(pl.program_id(0) != last)
    def _fast():
        out_ref[...] = compute(False)

    @pl.when(pl.program_id(0) == last)
    def _tail():
        out_ref[...] = compute(True)


@jax.jit
def _tc_pack(tableT):
    grid = (NSUP_PAD // SBLK,)
    return pl.pallas_call(
        _tc_pack_body,
        grid=grid,
        in_specs=[pl.BlockSpec((DIM, SBLK * 2 * TC_C), lambda i: (0, i))],
        out_specs=pl.BlockSpec((SBLK * TC_C, PDIM), lambda i: (i, 0)),
        out_shape=jax.ShapeDtypeStruct((PROWS, PDIM), jnp.float32),
    )(tableT)


def _sc_gather_body(tblv, relv, lhs_idx, rhs_idx, rel_idx, neg_idx,
                    lhs_out, rhs_out, ops_out, neg_out,
                    lhs_iv, rhs_iv, rel_iv, neg_iv,
                    lhs_rv, rhs_rv, ops_rv, neg_rv, sem):
    wid = lax.axis_index("s") * NC + lax.axis_index("c")
    base = wid * ROWS_W

    pltpu.sync_copy(lhs_idx.at[pl.ds(wid * NCHUNK, NCHUNK)], lhs_iv)
    pltpu.sync_copy(rhs_idx.at[pl.ds(wid * NCHUNK, NCHUNK)], rhs_iv)
    pltpu.sync_copy(rel_idx.at[pl.ds(wid * NCHUNK, NCHUNK)], rel_iv)
    pltpu.sync_copy(neg_idx.at[pl.ds(wid * NEG_W, NEG_W)], neg_iv)

    def _to_row(iv, n):
        for t in range(n // 16):
            sl = pl.ds(t * 16, 16)
            v = iv[sl]
            r = lax.bitwise_and(v, 2047)
            iv[sl] = ((v - r) + lax.shift_left(lax.bitwise_and(r, 1023), 1)
                      + lax.shift_right_logical(r, 10))

    for j in range(NCHUNK):
        _to_row(lhs_iv.at[j], CHUNK)
        _to_row(rhs_iv.at[j], CHUNK)
    _to_row(neg_iv, NEG_W)

    copies = []
    for j in range(NCHUNK):
        dst = pl.ds(j * CHUNK, CHUNK)
        copies.append(pltpu.async_copy(tblv.at[lhs_iv.at[j]],
                                       lhs_rv.at[dst], sem))
        copies.append(pltpu.async_copy(tblv.at[rhs_iv.at[j]],
                                       rhs_rv.at[dst], sem))
        copies.append(pltpu.async_copy(relv.at[rel_iv.at[j]],
                                       ops_rv.at[dst], sem))
    copies.append(pltpu.async_copy(tblv.at[neg_iv], neg_rv, sem))
    for c in copies:
        c.wait()

    pltpu.sync_copy(lhs_rv, lhs_out.at[pl.ds(base, ROWS_W)])
    pltpu.sync_copy(rhs_rv, rhs_out.at[pl.ds(base, ROWS_W)])
    pltpu.sync_copy(ops_rv, ops_out.at[pl.ds(base, ROWS_W)])
    pltpu.sync_copy(neg_rv, neg_out.at[pl.ds(wid * NEG_W, NEG_W)])


@jax.jit
def _sc_gather(tblv, relv, lhs_idx, rhs_idx, rel_idx, neg_idx):
    mesh = plsc.VectorSubcoreMesh(core_axis_name="c", subcore_axis_name="s",
                                  num_cores=NC, num_subcores=NS)
    f32 = jnp.float32
    return pl.kernel(
        _sc_gather_body,
        mesh=mesh,
        compiler_params=pltpu.CompilerParams(use_tc_tiling_on_sc=False),
        out_type=[
            jax.ShapeDtypeStruct((B, DIM), f32),
            jax.ShapeDtypeStruct((B, DIM), f32),
            jax.ShapeDtypeStruct((B, DIM), f32),
            jax.ShapeDtypeStruct((NNEG, DIM), f32),
        ],
        scratch_types=[
            pltpu.VMEM((NCHUNK, CHUNK), jnp.int32),
            pltpu.VMEM((NCHUNK, CHUNK), jnp.int32),
            pltpu.VMEM((NCHUNK, CHUNK), jnp.int32),
            pltpu.VMEM((NEG_W,), jnp.int32),
            pltpu.VMEM((ROWS_W, DIM), f32),
            pltpu.VMEM((ROWS_W, DIM), f32),
            pltpu.VMEM((ROWS_W, DIM), f32),
            pltpu.VMEM((NEG_W, DIM), f32),
            pltpu.SemaphoreType.DMA,
        ],
    )(tblv, relv, lhs_idx, rhs_idx, rel_idx, neg_idx)


def _tc_score_body(lhs_ref, rhs_ref, ops_ref, w2t_ref, out_ref):
    lhs = lhs_ref[...]
    ops = ops_ref[...]
    rhs_t = rhs_ref[...] + ops
    lops = jnp.sum(lhs * ops, axis=1, keepdims=True)
    pdot = jnp.sum(lhs * rhs_ref[...], axis=1, keepdims=True)
    x2 = jnp.concatenate([lhs, rhs_t, lops, pdot], axis=1)
    out_ref[...] = lax.dot_general(
        w2t_ref[...], x2, (((1,), (1,)), ((), ())),
        preferred_element_type=jnp.float32)


@jax.jit
def _tc_score(lhs, rhs, ops, w2t):
    grid = (B // BLK,)
    blk = pl.BlockSpec((BLK, DIM), lambda i: (i, 0))
    return pl.pallas_call(
        _tc_score_body,
        grid=grid,
        in_specs=[
            blk, blk, blk,
            pl.BlockSpec((NOUT, 2 * DIM + 2), lambda i: (0, 0)),
        ],
        out_specs=pl.BlockSpec((NOUT, BLK), lambda i: (0, i)),
        out_shape=jax.ShapeDtypeStruct((NOUT, B), jnp.float32),
    )(lhs, rhs, ops, w2t)


def kernel(lhs_idx, rhs_idx, rel_idx, neg_idx, table, rel_ops):
    tbl2 = _tc_pack(table.T)
    tblv = tbl2.reshape(2 * PROWS, DIM)

    li = lhs_idx.astype(jnp.int32)
    ri = rhs_idx.astype(jnp.int32)
    oi = rel_idx.astype(jnp.int32)
    ni = neg_idx.astype(jnp.int32)

    lhs_i2 = li.reshape(B // CHUNK, CHUNK)
    rhs_i2 = ri.reshape(B // CHUNK, CHUNK)
    rel_i2 = oi.reshape(B // CHUNK, CHUNK)
    neg_i = ni

    lhs, rhs, ops, neg = _sc_gather(tblv, rel_ops, lhs_i2, rhs_i2,
                                    rel_i2, neg_i)

    zn64 = jnp.zeros((NNEG, DIM), jnp.float32)
    on = jnp.ones((NNEG, 1), jnp.float32)
    zn1 = jnp.zeros((NNEG, 1), jnp.float32)
    row0 = jnp.concatenate([jnp.zeros((1, 2 * DIM), jnp.float32),
                            jnp.ones((1, 2), jnp.float32)], axis=1)
    blk_rhsneg = jnp.concatenate([neg, zn64, on, zn1], axis=1)
    blk_lhsneg = jnp.concatenate([zn64, neg, zn1, zn1], axis=1)
    w2t = jnp.concatenate([row0, blk_rhsneg, blk_lhsneg], axis=0)

    return _tc_score(lhs, rhs, ops, w2t).T
